# baseline (speedup 1.0000x reference)
import functools

import jax
import jax.numpy as jnp
from jax import lax
from jax.experimental import pallas as pl
from jax.experimental.pallas import tpu as pltpu
from jax.experimental.pallas import tpu_sc as plsc

B = 16384
D = 32
R = 1000000
L = 16
NC = 2
NS = 16
NW = NC * NS
BPW = B // NW
G = BPW // L
W128 = 128
DEPTH = 8


def _body(u_idx, i_idx, u_tab, i_tab, w_hbm, out_hbm,
          uixv, iixv,
          ub0, ub1, ub2, ub3, ub4, ub5, ub6, ub7,
          ib0, ib1, ib2, ib3, ib4, ib5, ib6, ib7,
          wv, pbuf, outv,
          su0, su1, su2, su3, su4, su5, su6, su7,
          si0, si1, si2, si3, si4, si5, si6, si7):
    wid = lax.axis_index("s") * NC + lax.axis_index("c")
    base = wid * BPW

    pltpu.sync_copy(u_idx.at[pl.ds(base, BPW)], uixv)
    pltpu.sync_copy(i_idx.at[pl.ds(base, BPW)], iixv)
    pltpu.sync_copy(w_hbm, wv)

    w_lo = wv[pl.ds(0, L)]
    w_hi = wv[pl.ds(L, L)]
    d_lo = lax.iota(jnp.int32, L)
    d_hi = d_lo + L
    lanes = lax.iota(jnp.int32, L)
    bufs = ((ub0, ib0, su0, si0), (ub1, ib1, su1, si1),
            (ub2, ib2, su2, si2), (ub3, ib3, su3, si3),
            (ub4, ib4, su4, si4), (ub5, ib5, su5, si5),
            (ub6, ib6, su6, si6), (ub7, ib7, su7, si7))

    def fire(ur, ir, p):
        ubuf, ibuf, usem, isem = bufs[p]
        uj = pl.multiple_of((ur >> 7) << 7, W128)
        ij = pl.multiple_of((ir >> 7) << 7, W128)
        pltpu.async_copy(u_tab.at[d_lo, pl.ds(uj, W128)],
                         ubuf.at[pl.ds(0, L), :], usem)
        pltpu.async_copy(u_tab.at[d_hi, pl.ds(uj, W128)],
                         ubuf.at[pl.ds(L, L), :], usem)
        pltpu.async_copy(i_tab.at[d_lo, pl.ds(ij, W128)],
                         ibuf.at[pl.ds(0, L), :], isem)
        pltpu.async_copy(i_tab.at[d_hi, pl.ds(ij, W128)],
                         ibuf.at[pl.ds(L, L), :], isem)

    def drain(p):
        ubuf, ibuf, usem, isem = bufs[p]
        dummy = u_tab.at[pl.ds(0, D), pl.ds(0, W128)]
        pltpu.make_async_copy(dummy, ubuf, usem).wait()
        pltpu.make_async_copy(dummy, ibuf, isem).wait()

    def compute(rr, ur, ir, p):
        ubuf, ibuf, _, _ = bufs[p]
        uq = jnp.broadcast_to(ur & 127, (L,))
        iq = jnp.broadcast_to(ir & 127, (L,))
        u_l = plsc.load_gather(ubuf, [d_lo, uq])
        u_h = plsc.load_gather(ubuf, [d_hi, uq])
        i_l = plsc.load_gather(ibuf, [d_lo, iq])
        i_h = plsc.load_gather(ibuf, [d_hi, iq])
        pbuf[rr, :] = u_l * i_l * w_lo + u_h * i_h * w_hi

    u0 = uixv[pl.ds(0, L)]
    i0 = iixv[pl.ds(0, L)]
    for k in range(DEPTH - 1):
        fire(u0[k], i0[k], k)

    def group(g, carry):
        cu = uixv[pl.ds(g * L, L)]
        ci = iixv[pl.ds(g * L, L)]
        gn = ((g + 1) & (G - 1)) * L
        nu = uixv[pl.ds(gn, L)]
        ni = iixv[pl.ds(gn, L)]
        for rr in range(L):
            tgt = rr + DEPTH - 1
            if tgt < L:
                fire(cu[tgt], ci[tgt], tgt & (DEPTH - 1))
            else:
                @pl.when(g + 1 < G)
                def _():
                    fire(nu[tgt - L], ni[tgt - L], tgt & (DEPTH - 1))
            drain(rr & (DEPTH - 1))
            compute(rr, cu[rr], ci[rr], rr & (DEPTH - 1))
        acc = jnp.zeros((L,), jnp.float32)
        for j in range(L):
            acc = acc + plsc.load_gather(
                pbuf, [lanes, jnp.full((L,), j, jnp.int32)])
        outv[pl.ds(g * L, L)] = 1.0 / (1.0 + jnp.exp(-acc))
        return carry

    lax.fori_loop(0, G, group, 0)

    pltpu.sync_copy(outv, out_hbm.at[pl.ds(base, BPW)])


_mesh = plsc.VectorSubcoreMesh(core_axis_name="c", subcore_axis_name="s")

_sc_call = functools.partial(
    pl.kernel,
    mesh=_mesh,
    compiler_params=pltpu.CompilerParams(needs_layout_passes=False),
    out_type=jax.ShapeDtypeStruct((B,), jnp.float32),
    scratch_types=[
        pltpu.VMEM((BPW,), jnp.int32),
        pltpu.VMEM((BPW,), jnp.int32),
        pltpu.VMEM((D, W128), jnp.float32),
        pltpu.VMEM((D, W128), jnp.float32),
        pltpu.VMEM((D, W128), jnp.float32),
        pltpu.VMEM((D, W128), jnp.float32),
        pltpu.VMEM((D, W128), jnp.float32),
        pltpu.VMEM((D, W128), jnp.float32),
        pltpu.VMEM((D, W128), jnp.float32),
        pltpu.VMEM((D, W128), jnp.float32),
        pltpu.VMEM((D, W128), jnp.float32),
        pltpu.VMEM((D, W128), jnp.float32),
        pltpu.VMEM((D, W128), jnp.float32),
        pltpu.VMEM((D, W128), jnp.float32),
        pltpu.VMEM((D, W128), jnp.float32),
        pltpu.VMEM((D, W128), jnp.float32),
        pltpu.VMEM((D, W128), jnp.float32),
        pltpu.VMEM((D, W128), jnp.float32),

        pltpu.VMEM((D,), jnp.float32),
        pltpu.VMEM((L, L), jnp.float32),
        pltpu.VMEM((BPW,), jnp.float32),
        pltpu.SemaphoreType.DMA,
        pltpu.SemaphoreType.DMA,
        pltpu.SemaphoreType.DMA,
        pltpu.SemaphoreType.DMA,
        pltpu.SemaphoreType.DMA,
        pltpu.SemaphoreType.DMA,
        pltpu.SemaphoreType.DMA,
        pltpu.SemaphoreType.DMA,
        pltpu.SemaphoreType.DMA,
        pltpu.SemaphoreType.DMA,
        pltpu.SemaphoreType.DMA,
        pltpu.SemaphoreType.DMA,
        pltpu.SemaphoreType.DMA,
        pltpu.SemaphoreType.DMA,
        pltpu.SemaphoreType.DMA,
        pltpu.SemaphoreType.DMA,
    ],
)(_body)


def kernel(user_indices, item_indices, user_table_gmf, item_table_gmf, W_gmf):
    ui = user_indices.astype(jnp.int32)
    ii = item_indices.astype(jnp.int32)
    ut = jnp.swapaxes(user_table_gmf, 0, 1)
    it = jnp.swapaxes(item_table_gmf, 0, 1)
    w = jnp.reshape(W_gmf, (D,))
    return _sc_call(ui, ii, ut, it, w)
